# Initial kernel scaffold; baseline (speedup 1.0000x reference)
import jax
import jax.numpy as jnp
from jax.experimental import pallas as pl


def kernel(x, y, row_table, col_table):
    raise NotImplementedError("write your pallas kernel here")


# speedup vs baseline: 2.7987x; 2.7987x over previous
import functools

import jax
import jax.numpy as jnp
from jax import lax
from jax.experimental import pallas as pl
from jax.experimental.pallas import tpu as pltpu
from jax.experimental.pallas import tpu_sc as plsc

_B = 16384
_D = 128
_NC = 2
_NS = 16
_NW = _NC * _NS
_BPW = _B // _NW


def _body(row_t, col_t, xi, yi, out_r, out_c, idx_v, rows_v, sem):
    wid = lax.axis_index("s") * _NC + lax.axis_index("c")
    base = wid * _BPW
    pltpu.sync_copy(xi.at[pl.ds(base, _BPW)], idx_v)
    pltpu.async_copy(row_t.at[idx_v], rows_v, sem).wait()
    pltpu.sync_copy(rows_v, out_r.at[pl.ds(base, _BPW)])
    pltpu.sync_copy(yi.at[pl.ds(base, _BPW)], idx_v)
    pltpu.async_copy(col_t.at[idx_v], rows_v, sem).wait()
    pltpu.sync_copy(rows_v, out_c.at[pl.ds(base, _BPW)])


_gather2 = functools.partial(
    pl.kernel,
    mesh=plsc.VectorSubcoreMesh(core_axis_name="c", subcore_axis_name="s"),
    out_type=(
        jax.ShapeDtypeStruct((_B, _D), jnp.float32),
        jax.ShapeDtypeStruct((_B, _D), jnp.float32),
    ),
    scratch_types=[
        pltpu.VMEM((_BPW,), jnp.int32),
        pltpu.VMEM((_BPW, _D), jnp.float32),
        pltpu.SemaphoreType.DMA,
    ],
)(_body)


@jax.jit
def kernel(x, y, row_table, col_table):
    xf = x.reshape(-1).astype(jnp.int32)
    yf = y.reshape(-1).astype(jnp.int32)
    return _gather2(row_table, col_table, xf, yf)


# speedup vs baseline: 2.8120x; 1.0047x over previous
import functools

import jax
import jax.numpy as jnp
from jax import lax
from jax.experimental import pallas as pl
from jax.experimental.pallas import tpu as pltpu
from jax.experimental.pallas import tpu_sc as plsc

_B = 16384
_D = 128
_NC = 2
_NS = 16
_NW = _NC * _NS
_BPW = _B // _NW
_CH = 128
_NCH = _BPW // _CH
_T = 2 * _NCH
_NB = 4


def _body(row_t, col_t, xi, yi, out_r, out_c,
          idx_x, idx_y, bufs, gs0, gs1, gs2, gs3, ws0, ws1, ws2, ws3):
    gsems = (gs0, gs1, gs2, gs3)
    wsems = (ws0, ws1, ws2, ws3)
    wid = lax.axis_index("s") * _NC + lax.axis_index("c")
    base = wid * _BPW
    pltpu.sync_copy(xi.at[pl.ds(base, _BPW)], idx_x)
    pltpu.sync_copy(yi.at[pl.ds(base, _BPW)], idx_y)

    def task(t):
        if t < _NCH:
            return row_t, idx_x, t * _CH, out_r
        return col_t, idx_y, (t - _NCH) * _CH, out_c

    gh = [None] * _T
    wh = [None] * _T
    for t in range(_T):
        b = t % _NB
        if t >= _NB:
            wh[t - _NB].wait()
        tab, idx, off, _ = task(t)
        gh[t] = pltpu.async_copy(tab.at[idx.at[pl.ds(off, _CH)]],
                                 bufs.at[b], gsems[b])
        d = t - (_NB - 1)
        if d >= 0:
            _, _, doff, dout = task(d)
            gh[d].wait()
            wh[d] = pltpu.async_copy(bufs.at[d % _NB],
                                     dout.at[pl.ds(base + doff, _CH)],
                                     wsems[d % _NB])
    for d in range(_T - (_NB - 1), _T):
        _, _, doff, dout = task(d)
        gh[d].wait()
        wh[d] = pltpu.async_copy(bufs.at[d % _NB],
                                 dout.at[pl.ds(base + doff, _CH)],
                                 wsems[d % _NB])
    for d in range(_T - _NB, _T):
        wh[d].wait()


_gather2 = functools.partial(
    pl.kernel,
    mesh=plsc.VectorSubcoreMesh(core_axis_name="c", subcore_axis_name="s"),
    out_type=(
        jax.ShapeDtypeStruct((_B, _D), jnp.float32),
        jax.ShapeDtypeStruct((_B, _D), jnp.float32),
    ),
    scratch_types=[
        pltpu.VMEM((_BPW,), jnp.int32),
        pltpu.VMEM((_BPW,), jnp.int32),
        pltpu.VMEM((_NB, _CH, _D), jnp.float32),
        pltpu.SemaphoreType.DMA,
        pltpu.SemaphoreType.DMA,
        pltpu.SemaphoreType.DMA,
        pltpu.SemaphoreType.DMA,
        pltpu.SemaphoreType.DMA,
        pltpu.SemaphoreType.DMA,
        pltpu.SemaphoreType.DMA,
        pltpu.SemaphoreType.DMA,
    ],
)(_body)


@jax.jit
def kernel(x, y, row_table, col_table):
    xf = x.reshape(-1).astype(jnp.int32)
    yf = y.reshape(-1).astype(jnp.int32)
    return _gather2(row_table, col_table, xf, yf)


# speedup vs baseline: 2.8387x; 1.0095x over previous
import functools

import jax
import jax.numpy as jnp
from jax import lax
from jax.experimental import pallas as pl
from jax.experimental.pallas import tpu as pltpu
from jax.experimental.pallas import tpu_sc as plsc

_B = 16384
_D = 128
_NC = 2
_NS = 16
_NW = _NC * _NS
_BPW = _B // _NW
_CH = 64
_NCH = _BPW // _CH
_T = 2 * _NCH
_NB = 8


def _body(row_t, col_t, xi, yi, out_r, out_c, idx_x, idx_y, bufs, *sems):
    gsems = sems[:_NB]
    wsems = sems[_NB:]
    wid = lax.axis_index("s") * _NC + lax.axis_index("c")
    base = wid * _BPW
    pltpu.sync_copy(xi.at[pl.ds(base, _BPW)], idx_x)
    pltpu.sync_copy(yi.at[pl.ds(base, _BPW)], idx_y)

    def task(t):
        if t < _NCH:
            return row_t, idx_x, t * _CH, out_r
        return col_t, idx_y, (t - _NCH) * _CH, out_c

    gh = [None] * _T
    wh = [None] * _T
    for t in range(_T):
        b = t % _NB
        if t >= _NB:
            wh[t - _NB].wait()
        tab, idx, off, _ = task(t)
        gh[t] = pltpu.async_copy(tab.at[idx.at[pl.ds(off, _CH)]],
                                 bufs.at[b], gsems[b])
        d = t - (_NB - 1)
        if d >= 0:
            _, _, doff, dout = task(d)
            gh[d].wait()
            wh[d] = pltpu.async_copy(bufs.at[d % _NB],
                                     dout.at[pl.ds(base + doff, _CH)],
                                     wsems[d % _NB])
    for d in range(_T - (_NB - 1), _T):
        _, _, doff, dout = task(d)
        gh[d].wait()
        wh[d] = pltpu.async_copy(bufs.at[d % _NB],
                                 dout.at[pl.ds(base + doff, _CH)],
                                 wsems[d % _NB])
    for d in range(_T - _NB, _T):
        wh[d].wait()


_gather2 = functools.partial(
    pl.kernel,
    mesh=plsc.VectorSubcoreMesh(core_axis_name="c", subcore_axis_name="s"),
    out_type=(
        jax.ShapeDtypeStruct((_B, _D), jnp.float32),
        jax.ShapeDtypeStruct((_B, _D), jnp.float32),
    ),
    scratch_types=[
        pltpu.VMEM((_BPW,), jnp.int32),
        pltpu.VMEM((_BPW,), jnp.int32),
        pltpu.VMEM((_NB, _CH, _D), jnp.float32),
    ] + [pltpu.SemaphoreType.DMA] * (2 * _NB),
)(_body)


@jax.jit
def kernel(x, y, row_table, col_table):
    xf = x.reshape(-1).astype(jnp.int32)
    yf = y.reshape(-1).astype(jnp.int32)
    return _gather2(row_table, col_table, xf, yf)
